# baseline (speedup 1.0000x reference)
import functools

import jax
import jax.numpy as jnp
from jax import lax
from jax.experimental import pallas as pl
from jax.experimental.pallas import tpu as pltpu
from jax.experimental.pallas import tpu_sc as plsc

N = 10000
E = 320000
D = 128
H = 128

NC = 2
NS = 16
NW = NC * NS
EPW = E // NW
EB = 128
NB = 80
EPAD = NB * EB - EPW
NA = N + 8
NH = N + 16
RPS = 624
ZR = 24
NTAIL = NA - NS * RPS


def _sc_gather_scatter(x, srcf, dstf):
    mesh = plsc.VectorSubcoreMesh(core_axis_name="c", subcore_axis_name="s")

    @functools.partial(
        pl.kernel,
        out_type=(
            jax.ShapeDtypeStruct((NC, NA, D), jnp.float32),
            jax.ShapeDtypeStruct((NC, NS, NH), jnp.float32),
        ),
        mesh=mesh,
        compiler_params=pltpu.CompilerParams(needs_layout_passes=False),
        scratch_types=[
            [pltpu.VMEM((EB,), jnp.int32)] * 4,
            [pltpu.VMEM((EB,), jnp.int32)] * 4,
            [pltpu.VMEM((EB, D), jnp.float32)] * 2,
            pltpu.VMEM((ZR, D), jnp.float32),
            pltpu.VMEM((NH,), jnp.float32),
            pltpu.VMEM_SHARED((NA, D), jnp.float32),
            [pltpu.SemaphoreType.DMA] * 4,
            [pltpu.SemaphoreType.DMA] * 2,
            [pltpu.SemaphoreType.DMA] * 2,
        ],
    )
    def body(x_hbm, src_hbm, dst_hbm, out_hbm, deg_hbm,
             sidx, didx, rows, zbuf, hist, acc, sem_i, sem_g, sem_s):
        c = lax.axis_index("c")
        s = lax.axis_index("s")
        w = c * NS + s
        base = w * NB * EB

        def idx_start(j, r):
            pltpu.async_copy(src_hbm.at[pl.ds(base + j * EB, EB)], sidx[r], sem_i[r])
            pltpu.async_copy(dst_hbm.at[pl.ds(base + j * EB, EB)], didx[r], sem_i[r])

        def idx_wait(j, r):
            pltpu.make_async_copy(
                src_hbm.at[pl.ds(base + j * EB, EB)], sidx[r], sem_i[r]).wait()
            pltpu.make_async_copy(
                dst_hbm.at[pl.ds(base + j * EB, EB)], didx[r], sem_i[r]).wait()

        def gather_start(r, p):
            pltpu.async_copy(x_hbm.at[sidx[r]], rows[p], sem_g[p])

        def gather_wait(r, p):
            pltpu.make_async_copy(x_hbm.at[sidx[r]], rows[p], sem_g[p]).wait()

        def scatter_start(r, p):
            pltpu.async_copy(rows[p], acc.at[didx[r]], sem_s[p], add=True)

        def scatter_wait(r, p):
            pltpu.make_async_copy(rows[p], acc.at[didx[r]], sem_s[p]).wait()

        for r in range(4):
            idx_start(r, r)

        z16 = jnp.zeros((16,), jnp.float32)
        for r in range(ZR):
            for k in range(D // 16):
                zbuf[r, pl.ds(k * 16, 16)] = z16

        def zstep(j, carry):
            pltpu.sync_copy(zbuf, acc.at[pl.ds(s * RPS + j * ZR, ZR)])
            return carry

        lax.fori_loop(0, RPS // ZR, zstep, 0)

        @pl.when(s == 0)
        def _():
            pltpu.sync_copy(zbuf.at[pl.ds(0, NTAIL)], acc.at[pl.ds(NS * RPS, NTAIL)])

        def hzstep(j, carry):
            hist[pl.ds(j * 16, 16)] = z16
            return carry

        lax.fori_loop(0, NH // 16, hzstep, 0)
        plsc.subcore_barrier()

        ones16 = jnp.ones((16,), jnp.float32)

        def hist_adds(r):
            for k in range(EB // 16):
                idx16 = didx[r][pl.ds(k * 16, 16)]
                plsc.addupdate_scatter(hist, [idx16], ones16)

        idx_wait(0, 0)
        gather_start(0, 0)

        def qstep(q, carry):
            j0 = 4 * q

            gather_wait(0, 0)

            @pl.when(q > 0)
            def _():
                scatter_wait(3, 1)
                idx_start(j0 + 3, 3)

            scatter_start(0, 0)
            idx_wait(j0 + 1, 1)
            gather_start(1, 1)
            hist_adds(0)

            gather_wait(1, 1)
            scatter_wait(0, 0)
            @pl.when(j0 + 4 < NB)
            def _():
                idx_start(j0 + 4, 0)
            scatter_start(1, 1)
            idx_wait(j0 + 2, 2)
            gather_start(2, 0)
            hist_adds(1)

            gather_wait(2, 0)
            scatter_wait(1, 1)
            @pl.when(j0 + 5 < NB)
            def _():
                idx_start(j0 + 5, 1)
            scatter_start(2, 0)
            idx_wait(j0 + 3, 3)
            gather_start(3, 1)
            hist_adds(2)

            gather_wait(3, 1)
            scatter_wait(2, 0)
            @pl.when(j0 + 6 < NB)
            def _():
                idx_start(j0 + 6, 2)
            scatter_start(3, 1)

            @pl.when(j0 + 4 < NB)
            def _():
                idx_wait(j0 + 4, 0)
                gather_start(0, 0)
            hist_adds(3)
            return carry

        lax.fori_loop(0, NB // 4, qstep, 0)
        scatter_wait(3, 1)

        plsc.subcore_barrier()

        pltpu.sync_copy(hist, deg_hbm.at[c, s])

        @pl.when(s == 0)
        def _():
            pltpu.sync_copy(acc, out_hbm.at[c])

    return body(x, srcf, dstf)


BN = 1000


def _tc_body(x_ref, h_ref, c_ref, a0_ref, a1_ref, deg_ref, w1_ref, w2_ref,
             whh_ref, b_ref, wr_ref, br_ref, nh_ref, nc_ref, rp_ref):
    deg = jnp.maximum(deg_ref[...], 1.0)
    agg = (a0_ref[...] + a1_ref[...]) / deg

    xb = x_ref[...]
    hb = h_ref[...]
    gates = (
        jnp.dot(xb, w1_ref[...], preferred_element_type=jnp.float32)
        + jnp.dot(agg, w2_ref[...], preferred_element_type=jnp.float32)
        + jnp.dot(hb, whh_ref[...], preferred_element_type=jnp.float32)
        + b_ref[...]
    )
    i_g = 1.0 / (1.0 + jnp.exp(-gates[:, 0 * H:1 * H]))
    f_g = 1.0 / (1.0 + jnp.exp(-gates[:, 1 * H:2 * H]))
    g_g = jnp.tanh(gates[:, 2 * H:3 * H])
    o_g = 1.0 / (1.0 + jnp.exp(-gates[:, 3 * H:4 * H]))
    new_c = f_g * c_ref[...] + i_g * g_g
    new_h = o_g * jnp.tanh(new_c)
    nc_ref[...] = new_c
    nh_ref[...] = new_h

    logits = jnp.dot(new_h, wr_ref[...], preferred_element_type=jnp.float32)
    logits = logits + br_ref[...]
    m = jnp.max(logits, axis=1, keepdims=True)
    e = jnp.exp(logits - m)
    probs = e / jnp.sum(e, axis=1, keepdims=True)
    rp_ref[...] = probs[:, :3]


def _tc_dense(x, h, c, a0, a1, deg, W1, W2, W_hh, b, Wr_pad, br_pad):
    grid = (N // BN,)
    blk = lambda rows, cols: pl.BlockSpec((rows, cols), lambda i: (i, 0))
    full = lambda rows, cols: pl.BlockSpec((rows, cols), lambda i: (0, 0))
    return pl.pallas_call(
        _tc_body,
        grid=grid,
        in_specs=[
            blk(BN, D),
            blk(BN, H),
            blk(BN, H),
            blk(BN, D),
            blk(BN, D),
            blk(BN, 1),
            full(D, 4 * H),
            full(D, 4 * H),
            full(H, 4 * H),
            pl.BlockSpec((4 * H,), lambda i: (0,)),
            full(H, 128),
            pl.BlockSpec((128,), lambda i: (0,)),
        ],
        out_specs=[
            blk(BN, H),
            blk(BN, H),
            blk(BN, 3),
        ],
        out_shape=[
            jax.ShapeDtypeStruct((N, H), jnp.float32),
            jax.ShapeDtypeStruct((N, H), jnp.float32),
            jax.ShapeDtypeStruct((N, 3), jnp.float32),
        ],
    )(x, h, c, a0, a1, deg, W1, W2, W_hh, b, Wr_pad, br_pad)


def kernel(x, h, c, edge_index, W_ih, W_hh, b, W_role, b_role):
    src_w = edge_index[0].reshape(NW, EPW)
    dst_w = edge_index[1].reshape(NW, EPW)
    src3 = jnp.pad(src_w, ((0, 0), (0, EPAD))).reshape(NW * NB * EB)
    dst3 = jnp.pad(dst_w, ((0, 0), (0, EPAD)),
                   constant_values=N).reshape(NW * NB * EB)
    part, deg_tiles = _sc_gather_scatter(x, src3, dst3)
    deg = jnp.sum(deg_tiles, axis=(0, 1))[:N].reshape(N, 1)

    W1 = W_ih[:D]
    W2 = W_ih[D:]
    Wr_pad = jnp.zeros((H, 128), jnp.float32).at[:, :3].set(W_role)
    br_pad = jnp.full((128,), -1e30, jnp.float32).at[:3].set(b_role)

    new_h, new_c, role_probs = _tc_dense(
        x, h, c, part[0, :N], part[1, :N], deg, W1, W2, W_hh, b, Wr_pad, br_pad
    )
    return new_h, new_c, role_probs


# speedup vs baseline: 1.7380x; 1.7380x over previous
import functools

import jax
import jax.numpy as jnp
from jax import lax
from jax.experimental import pallas as pl
from jax.experimental.pallas import tpu as pltpu
from jax.experimental.pallas import tpu_sc as plsc

N = 10000
E = 320000
D = 128
H = 128

NC = 2
NS = 16
NW = NC * NS
EPW = E // NW
EB = 80
NB = EPW // EB
NQ = NB // 4
NA = N + 8
NH = N + 16
RPS = 624
ZR = 24
NTAIL = NA - NS * RPS


def _sc_gather_scatter(x, srcf, dstf):
    mesh = plsc.VectorSubcoreMesh(core_axis_name="c", subcore_axis_name="s")

    @functools.partial(
        pl.kernel,
        out_type=(
            jax.ShapeDtypeStruct((NC, NA, D), jnp.float32),
            jax.ShapeDtypeStruct((NC, NS, NH), jnp.float32),
        ),
        mesh=mesh,
        compiler_params=pltpu.CompilerParams(needs_layout_passes=False),
        scratch_types=[
            [pltpu.VMEM((EB,), jnp.int32)] * 4,
            [pltpu.VMEM((EB,), jnp.int32)] * 4,
            pltpu.VMEM((EB, D), jnp.float32),
            pltpu.VMEM((ZR, D), jnp.float32),
            pltpu.VMEM((NH,), jnp.float32),
            pltpu.VMEM_SHARED((NA, D), jnp.float32),
            [pltpu.SemaphoreType.DMA] * 4,
            pltpu.SemaphoreType.DMA,
        ],
    )
    def body(x_hbm, src_hbm, dst_hbm, out_hbm, deg_hbm,
             sidx, didx, rows, zbuf, hist, acc, sem_i, sem_g):
        c = lax.axis_index("c")
        s = lax.axis_index("s")
        w = c * NS + s
        base = w * EPW

        def idx_start(j, r):
            pltpu.async_copy(src_hbm.at[pl.ds(base + j * EB, EB)], sidx[r], sem_i[r])
            pltpu.async_copy(dst_hbm.at[pl.ds(base + j * EB, EB)], didx[r], sem_i[r])

        def idx_wait(j, r):
            pltpu.make_async_copy(
                src_hbm.at[pl.ds(base + j * EB, EB)], sidx[r], sem_i[r]).wait()
            pltpu.make_async_copy(
                dst_hbm.at[pl.ds(base + j * EB, EB)], didx[r], sem_i[r]).wait()

        for r in range(4):
            idx_start(r, r)

        z16 = jnp.zeros((16,), jnp.float32)
        for r in range(ZR):
            for k in range(D // 16):
                zbuf[r, pl.ds(k * 16, 16)] = z16

        def zstep(j, carry):
            pltpu.sync_copy(zbuf, acc.at[pl.ds(s * RPS + j * ZR, ZR)])
            return carry

        lax.fori_loop(0, RPS // ZR, zstep, 0)

        @pl.when(s == 0)
        def _():
            pltpu.sync_copy(zbuf.at[pl.ds(0, NTAIL)], acc.at[pl.ds(NS * RPS, NTAIL)])

        def hzstep(j, carry):
            hist[pl.ds(j * 16, 16)] = z16
            return carry

        lax.fori_loop(0, NH // 16, hzstep, 0)
        plsc.subcore_barrier()

        ones16 = jnp.ones((16,), jnp.float32)

        def hist_adds(r):
            for k in range(EB // 16):
                idx16 = didx[r][pl.ds(k * 16, 16)]
                plsc.addupdate_scatter(hist, [idx16], ones16)

        def do_batch(j, r):
            idx_wait(j, r)
            pltpu.async_copy(x_hbm.at[sidx[r]], rows, sem_g).wait()
            pltpu.sync_copy(rows, acc.at[didx[r]], add=True)
            hist_adds(r)

            @pl.when(j + 4 < NB)
            def _():
                idx_start(j + 4, r)

        def qstep(q, carry):
            j0 = 4 * q
            do_batch(j0, 0)
            do_batch(j0 + 1, 1)
            do_batch(j0 + 2, 2)
            do_batch(j0 + 3, 3)
            return carry

        lax.fori_loop(0, NQ, qstep, 0)
        for i in range(NQ * 4, NB):
            do_batch(i, i % 4)

        plsc.subcore_barrier()

        pltpu.sync_copy(hist, deg_hbm.at[c, s])

        @pl.when(s == 0)
        def _():
            pltpu.sync_copy(acc, out_hbm.at[c])

    return body(x, srcf, dstf)


BN = 1000


def _tc_body(x_ref, h_ref, c_ref, a0_ref, a1_ref, deg_ref, w1_ref, w2_ref,
             whh_ref, b_ref, wr_ref, br_ref, nh_ref, nc_ref, rp_ref):
    deg = jnp.maximum(deg_ref[...], 1.0)
    agg = (a0_ref[...] + a1_ref[...]) / deg

    xb = x_ref[...]
    hb = h_ref[...]
    gates = (
        jnp.dot(xb, w1_ref[...], preferred_element_type=jnp.float32)
        + jnp.dot(agg, w2_ref[...], preferred_element_type=jnp.float32)
        + jnp.dot(hb, whh_ref[...], preferred_element_type=jnp.float32)
        + b_ref[...]
    )
    i_g = 1.0 / (1.0 + jnp.exp(-gates[:, 0 * H:1 * H]))
    f_g = 1.0 / (1.0 + jnp.exp(-gates[:, 1 * H:2 * H]))
    g_g = jnp.tanh(gates[:, 2 * H:3 * H])
    o_g = 1.0 / (1.0 + jnp.exp(-gates[:, 3 * H:4 * H]))
    new_c = f_g * c_ref[...] + i_g * g_g
    new_h = o_g * jnp.tanh(new_c)
    nc_ref[...] = new_c
    nh_ref[...] = new_h

    logits = jnp.dot(new_h, wr_ref[...], preferred_element_type=jnp.float32)
    logits = logits + br_ref[...]
    m = jnp.max(logits, axis=1, keepdims=True)
    e = jnp.exp(logits - m)
    probs = e / jnp.sum(e, axis=1, keepdims=True)
    rp_ref[...] = probs[:, :3]


def _tc_dense(x, h, c, a0, a1, deg, W1, W2, W_hh, b, Wr_pad, br_pad):
    grid = (N // BN,)
    blk = lambda rows, cols: pl.BlockSpec((rows, cols), lambda i: (i, 0))
    full = lambda rows, cols: pl.BlockSpec((rows, cols), lambda i: (0, 0))
    return pl.pallas_call(
        _tc_body,
        grid=grid,
        in_specs=[
            blk(BN, D),
            blk(BN, H),
            blk(BN, H),
            blk(BN, D),
            blk(BN, D),
            blk(BN, 1),
            full(D, 4 * H),
            full(D, 4 * H),
            full(H, 4 * H),
            pl.BlockSpec((4 * H,), lambda i: (0,)),
            full(H, 128),
            pl.BlockSpec((128,), lambda i: (0,)),
        ],
        out_specs=[
            blk(BN, H),
            blk(BN, H),
            blk(BN, 3),
        ],
        out_shape=[
            jax.ShapeDtypeStruct((N, H), jnp.float32),
            jax.ShapeDtypeStruct((N, H), jnp.float32),
            jax.ShapeDtypeStruct((N, 3), jnp.float32),
        ],
    )(x, h, c, a0, a1, deg, W1, W2, W_hh, b, Wr_pad, br_pad)


def kernel(x, h, c, edge_index, W_ih, W_hh, b, W_role, b_role):
    src3 = edge_index[0]
    dst3 = edge_index[1]
    part, deg_tiles = _sc_gather_scatter(x, src3, dst3)
    deg = jnp.sum(deg_tiles, axis=(0, 1))[:N].reshape(N, 1)

    W1 = W_ih[:D]
    W2 = W_ih[D:]
    Wr_pad = jnp.zeros((H, 128), jnp.float32).at[:, :3].set(W_role)
    br_pad = jnp.full((128,), -1e30, jnp.float32).at[:3].set(b_role)

    new_h, new_c, role_probs = _tc_dense(
        x, h, c, part[0, :N], part[1, :N], deg, W1, W2, W_hh, b, Wr_pad, br_pad
    )
    return new_h, new_c, role_probs
